# baseline (speedup 1.0000x reference)
import jax
import jax.numpy as jnp
from jax.experimental import pallas as pl
from jax.experimental.pallas import tpu as pltpu


def _linear_t2_kernel(xa_ref, xb_ref, w_ref, b_ref, o_ref):
    wb = w_ref[...].astype(jnp.bfloat16)
    tb = xa_ref.shape[0]
    for k, x_ref in enumerate((xa_ref, xb_ref)):
        xv = x_ref[...].astype(jnp.bfloat16)
        y = jax.lax.dot_general(
            xv, wb, (((1,), (1,)), ((), ())),
            preferred_element_type=jnp.float32,
        )
        o_ref[:, k * tb:(k + 1) * tb] = y.T + b_ref[...]


_BATCH_TILE = 16384


def kernel(x, weight, bias, *, batch_tile=_BATCH_TILE):
    B, D_in = x.shape
    D_out = weight.shape[0]

    b2d = bias.reshape(D_out, 1)

    tb = min(batch_tile, B // 2)
    cost = pl.CostEstimate(
        flops=2 * B * D_in * D_out,
        transcendentals=0,
        bytes_accessed=(x.size + B * D_out + weight.size + b2d.size) * 4,
    )

    y_t = pl.pallas_call(
        _linear_t2_kernel,
        out_shape=jax.ShapeDtypeStruct((D_out, B), x.dtype),
        grid=(pl.cdiv(B, 2 * tb),),
        in_specs=[
            pl.BlockSpec((tb, D_in), lambda i: (2 * i, 0)),
            pl.BlockSpec((tb, D_in), lambda i: (2 * i + 1, 0)),
            pl.BlockSpec((D_out, D_in), lambda i: (0, 0)),
            pl.BlockSpec((D_out, 1), lambda i: (0, 0)),
        ],
        out_specs=pl.BlockSpec((D_out, 2 * tb), lambda i: (0, i)),
        compiler_params=pltpu.CompilerParams(
            dimension_semantics=("parallel",),
        ),
        cost_estimate=cost,
    )(x, x, weight, b2d)
    return y_t.T


# speedup vs baseline: 1.0544x; 1.0544x over previous
import jax
import jax.numpy as jnp
from jax.experimental import pallas as pl
from jax.experimental.pallas import tpu as pltpu


def _linear_t_kernel(x_ref, w_ref, b_ref, o_ref):
    xb = x_ref[...].astype(jnp.bfloat16)
    wb = w_ref[...].astype(jnp.bfloat16)
    y = jax.lax.dot_general(
        xb, wb, (((1,), (1,)), ((), ())),
        preferred_element_type=jnp.float32,
    )
    o_ref[...] = y.T + b_ref[...]


_BATCH_TILE = 16384


def kernel(x, weight, bias, *, batch_tile=_BATCH_TILE):
    B, D_in = x.shape
    D_out = weight.shape[0]

    b2d = bias.reshape(D_out, 1)

    tb = min(batch_tile, B)
    cost = pl.CostEstimate(
        flops=2 * B * D_in * D_out,
        transcendentals=0,
        bytes_accessed=(x.size + B * D_out + weight.size + b2d.size) * 4,
    )

    y_t = pl.pallas_call(
        _linear_t_kernel,
        out_shape=jax.ShapeDtypeStruct((D_out, B), x.dtype),
        grid=(pl.cdiv(B, tb),),
        in_specs=[
            pl.BlockSpec((tb, D_in), lambda i: (i, 0)),
            pl.BlockSpec((D_out, D_in), lambda i: (0, 0)),
            pl.BlockSpec((D_out, 1), lambda i: (0, 0)),
        ],
        out_specs=pl.BlockSpec((D_out, tb), lambda i: (0, i)),
        compiler_params=pltpu.CompilerParams(
            dimension_semantics=("parallel",),
        ),
        cost_estimate=cost,
    )(x, weight, b2d)
    return y_t.T
